# baseline (speedup 1.0000x reference)
import math

import jax
import jax.numpy as jnp
from jax import lax
from jax.experimental import pallas as pl
from jax.experimental.pallas import tpu as pltpu
from jax.experimental.pallas import tpu_sc as plsc

S = 64.0
M1 = 0.6
M2 = 0.4

_NW = 32
_NB = 8
_PF = 4
_CW = 1536
_CR = 8
_VBULK = 99840
_NCC = _VBULK // _CW
_UN = 4


def _asin_core(x, s, small):
    z = jnp.where(small, x * x, 0.5 * (1.0 - x))
    p = z * (0.16666666666666666 + z * (-0.3255658186224009 + z * (
        0.20121253213486293 + z * (-0.04005553450067941 + z * (
            7.915349942898145e-4 + z * 3.4793310759602117e-5)))))
    q = 1.0 + z * (-2.403394911734414 + z * (2.0209457602335057 + z * (
        -0.688283971605453 + z * 0.07703815055590194)))
    r = p / q
    u = jnp.where(small, x, s)
    asin_u = u + u * r
    return jnp.where(small, asin_u, 0.5 * math.pi - 2.0 * asin_u)


def _tan_small(y):
    y2 = y * y
    sin_y = y * (1.0 + y2 * (-1.0 / 6.0 + y2 * (1.0 / 120.0 + y2 * (
        -1.0 / 5040.0 + y2 * (1.0 / 362880.0)))))
    cos_y = 1.0 + y2 * (-0.5 + y2 * (1.0 / 24.0 + y2 * (-1.0 / 720.0 + y2 * (
        1.0 / 40320.0 + y2 * (-1.0 / 3628800.0)))))
    return sin_y / cos_y


def _margin_tc(x):
    s = jnp.sqrt(0.5 * (1.0 - x))
    t = _asin_core(x, s, x < 0.5)
    return _tan_small(M1 * t) - M2


def _sqrt_newton(w):
    wi = lax.bitcast_convert_type(w, jnp.int32)
    y = lax.bitcast_convert_type(
        (wi >> 1) + jnp.int32(0x1FBD1DF6), jnp.float32)
    for _ in range(4):
        y = 0.5 * (y + w / y)
    return y


def _margin_sc(x):
    s = _sqrt_newton(0.5 * (1.0 - x))
    t = _asin_core(x, s, x < 0.5)
    return _tan_small(M1 * t) - M2


def _sc_body(x_hbm, lab_hbm, out_hbm,
             b0, b1, b2, b3, b4, b5, b6, b7, lab_v,
             si0, si1, si2, si3, si4, si5, si6, si7,
             so0, so1, so2, so3, so4, so5, so6, so7):
    n_rows = x_hbm.shape[0]
    rpw = n_rows // _NW
    nch = (rpw // _CR) * _NCC

    wid = lax.axis_index("s") * 2 + lax.axis_index("c")
    base = wid * rpw

    nslot = (rpw // _CR) * 16
    pltpu.sync_copy(lab_hbm.at[pl.ds(wid * nslot, nslot)], lab_v)

    bufs = (b0, b1, b2, b3, b4, b5, b6, b7)
    sems_i = (si0, si1, si2, si3, si4, si5, si6, si7)
    sems_o = (so0, so1, so2, so3, so4, so5, so6, so7)

    def slc(i):
        rg = lax.div(i, _NCC)
        cc = lax.rem(i, _NCC)
        r0 = pl.multiple_of(base + rg * _CR, _CR)
        c0 = pl.multiple_of(cc * _CW, 128)
        return (pl.ds(r0, _CR), pl.ds(c0, _CW))

    def in_cp(i, b):
        r, c = slc(i)
        return pltpu.make_async_copy(x_hbm.at[r, c], bufs[b], sems_i[b])

    def out_cp(i, b):
        r, c = slc(i)
        return pltpu.make_async_copy(bufs[b], out_hbm.at[r, c], sems_o[b])

    for k in range(_PF):
        in_cp(k, k).start()

    def qstep(q, carry):
        for b in range(_NB):
            i = _NB * q + b
            in_cp(i, b).wait()

            rg = lax.div(i, _NCC)
            c0 = lax.rem(i, _NCC) * _CW

            lv = lab_v[pl.ds(pl.multiple_of(rg * 16, 16), 16)]
            for r in range(_CR):
                lab = lv[r].astype(jnp.int32)
                rel = lab - c0
                pred = (lab >= 0) & (rel >= 0) & (rel < _CW)

                @pl.when(pred)
                def _(_b=b, _r=r, _rel=rel):
                    start = pl.multiple_of(lax.div(_rel, 16) * 16, 16)
                    lane = lax.rem(_rel, 16)
                    xv = bufs[_b][_r, pl.ds(start, 16)]
                    mv = _margin_sc(xv)
                    sel = lax.iota(jnp.int32, 16) == lane
                    bufs[_b][_r, pl.ds(start, 16)] = jnp.where(sel, mv, xv)

            def vstep(j, c2, _b=b):
                for r in range(_CR):
                    for u in range(_UN):
                        off = pl.multiple_of(j * (16 * _UN) + u * 16, 16)
                        bufs[_b][r, pl.ds(off, 16)] = (
                            bufs[_b][r, pl.ds(off, 16)] * S)
                return c2

            lax.fori_loop(0, _CW // (16 * _UN), vstep, 0)

            out_cp(i, b).start()

            @pl.when(i >= _NB - _PF)
            def _(_i=i, _b=b):
                out_cp(_i - (_NB - _PF), (_b + _PF) % _NB).wait()

            @pl.when(i + _PF < nch)
            def _(_i=i, _b=b):
                in_cp(_i + _PF, (_b + _PF) % _NB).start()

        return carry

    lax.fori_loop(0, nch // _NB, qstep, 0)
    for k in range(_NB - _PF, 0, -1):
        out_cp(nch - k, (nch - k) % _NB).wait()


def _tc_tail(lab_ref, x_ref, bulk_ref, o_ref, scr, sem):
    del bulk_ref
    lab = lab_ref[0, 0, :]
    rel = jnp.where(lab >= 0, lab, -1) - _VBULK
    x = x_ref[...]
    col = lax.broadcasted_iota(jnp.int32, x.shape, 1)
    match = col == rel[:, None]
    target = jnp.sum(jnp.where(match, x, 0.0), axis=1)
    newv = _margin_tc(target)
    scr[...] = jnp.where(match, newv[:, None], x) * S
    vt = x.shape[1]
    pltpu.make_async_copy(
        scr, o_ref.at[:, pl.ds(_VBULK, vt)], sem).start()
    pltpu.make_async_copy(
        scr, o_ref.at[:, pl.ds(_VBULK, vt)], sem).wait()


@jax.jit
def kernel(logits, labels):
    B, V = logits.shape
    vt = V - _VBULK
    lab32 = labels.astype(jnp.int32)

    mesh = plsc.VectorSubcoreMesh(core_axis_name="c", subcore_axis_name="s")
    sc_fn = pl.kernel(
        _sc_body,
        out_type=jax.ShapeDtypeStruct((B, V), jnp.float32),
        mesh=mesh,
        scratch_types=(
            [pltpu.VMEM((_CR, _CW), jnp.float32) for _ in range(_NB)]
            + [pltpu.VMEM(((B // _NW // _CR) * 16,), jnp.float32)]
            + [pltpu.SemaphoreType.DMA for _ in range(2 * _NB)]
        ),
    )
    labf = labels.astype(jnp.float32).reshape(B // _CR, _CR)
    lab_pad = jnp.full((B // _CR, 16), -1.0, jnp.float32)
    lab_pad = lab_pad.at[:, :_CR].set(labf).reshape(-1)
    bulk = sc_fn(logits, lab_pad)

    lab3 = lab32.reshape(1, 1, B)
    tail_in = lax.slice(logits, (0, _VBULK), (B, V))
    return pl.pallas_call(
        _tc_tail,
        grid=(1,),
        in_specs=[
            pl.BlockSpec((1, 1, B), lambda i: (0, 0, 0)),
            pl.BlockSpec((B, vt), lambda i: (0, 0)),
            pl.BlockSpec(memory_space=pltpu.MemorySpace.HBM),
        ],
        out_specs=pl.BlockSpec(memory_space=pltpu.MemorySpace.HBM),
        out_shape=jax.ShapeDtypeStruct((B, V), jnp.float32),
        input_output_aliases={2: 0},
        scratch_shapes=[
            pltpu.VMEM((B, vt), jnp.float32),
            pltpu.SemaphoreType.DMA,
        ],
    )(lab3, tail_in, bulk)


# speedup vs baseline: 1.4662x; 1.4662x over previous
import math

import jax
import jax.numpy as jnp
from jax.experimental import pallas as pl
from jax.experimental.pallas import tpu as pltpu

S = 64.0
M1 = 0.6
M2 = 0.4

_RCH = 8
_NBUF = 8


def _asin01(x):
    z_small = x * x
    w = 0.5 * (1.0 - x)
    s = jnp.sqrt(w)
    small = x < 0.5
    z = jnp.where(small, z_small, w)
    p = z * (0.16666666666666666 + z * (-0.3255658186224009 + z * (
        0.20121253213486293 + z * (-0.04005553450067941 + z * (
            7.915349942898145e-4 + z * 3.4793310759602117e-5)))))
    q = 1.0 + z * (-2.403394911734414 + z * (2.0209457602335057 + z * (
        -0.688283971605453 + z * 0.07703815055590194)))
    r = p / q
    u = jnp.where(small, x, s)
    asin_u = u + u * r
    return jnp.where(small, asin_u, 0.5 * math.pi - 2.0 * asin_u)


def _tan_small(y):
    y2 = y * y
    sin_y = y * (1.0 + y2 * (-1.0 / 6.0 + y2 * (1.0 / 120.0 + y2 * (
        -1.0 / 5040.0 + y2 * (1.0 / 362880.0)))))
    cos_y = 1.0 + y2 * (-0.5 + y2 * (1.0 / 24.0 + y2 * (-1.0 / 720.0 + y2 * (
        1.0 / 40320.0 + y2 * (-1.0 / 3628800.0)))))
    return sin_y / cos_y


def _body(lab_ref, x_hbm, out_hbm, in_buf, out_buf, in_sems, out_sems):
    n_rows, v = x_hbm.shape
    nch = n_rows // _RCH

    def in_copy(i, b):
        return pltpu.make_async_copy(
            x_hbm.at[pl.ds(i * _RCH, _RCH)], in_buf.at[b], in_sems.at[b])

    def out_copy(i, b):
        return pltpu.make_async_copy(
            out_buf.at[b], out_hbm.at[pl.ds(i * _RCH, _RCH)], out_sems.at[b])

    for b in range(_NBUF):
        in_copy(b, b).start()

    def step(i, carry):
        b = jax.lax.rem(i, _NBUF)
        in_copy(i, b).wait()

        @pl.when(i >= _NBUF)
        def _():
            out_copy(i - _NBUF, b).wait()

        x = in_buf[b]
        lab = lab_ref[i]
        lab_s = jnp.where(lab >= 0, lab, -1)
        col = jax.lax.broadcasted_iota(jnp.int32, (_RCH, v), 1)
        match = col == lab_s[:, None]
        target = jnp.sum(jnp.where(match, x, 0.0), axis=1)
        newv = _tan_small(M1 * _asin01(target)) - M2
        out_buf[b] = jnp.where(match, newv[:, None], x) * S

        out_copy(i, b).start()

        @pl.when(i + _NBUF < nch)
        def _():
            in_copy(i + _NBUF, b).start()

        return carry

    jax.lax.fori_loop(0, nch, step, 0)

    def drain(i, carry):
        b = jax.lax.rem(i, _NBUF)
        out_copy(i, b).wait()
        return carry

    jax.lax.fori_loop(nch - _NBUF, nch, drain, 0)


@jax.jit
def kernel(logits, labels):
    B, V = logits.shape
    nch = B // _RCH
    lab2 = labels.astype(jnp.int32).reshape(nch, _RCH)
    return pl.pallas_call(
        _body,
        in_specs=[
            pl.BlockSpec(memory_space=pltpu.MemorySpace.VMEM),
            pl.BlockSpec(memory_space=pltpu.MemorySpace.HBM),
        ],
        out_specs=pl.BlockSpec(memory_space=pltpu.MemorySpace.HBM),
        out_shape=jax.ShapeDtypeStruct((B, V), jnp.float32),
        scratch_shapes=[
            pltpu.VMEM((_NBUF, _RCH, V), jnp.float32),
            pltpu.VMEM((_NBUF, _RCH, V), jnp.float32),
            pltpu.SemaphoreType.DMA((_NBUF,)),
            pltpu.SemaphoreType.DMA((_NBUF,)),
        ],
    )(lab2, logits)
